# baseline (speedup 1.0000x reference)
import jax
import jax.numpy as jnp
from jax.experimental import pallas as pl
from jax.experimental.pallas import tpu as pltpu

_B = 4096
_D = 128
_C = 512
_NB = 1
_BM = _B // _NB


def _fold_rows(v):
    r = v.shape[0]
    while r > 8:
        h = r // 2
        v = v[:h, :] + v[h:, :]
        r = h
    return v


def _dist_kernel(x_ref, k_ref, w_ref, e_ref, out0_ref, out1_ref,
                 ww_ref, ew2_ref, ees_ref,
                 arelu_ref, acnt_ref, amk_ref, axx_ref):
    i = pl.program_id(0)
    s = 1.0 / _D

    @pl.when(i == 0)
    def _prep():
        w = w_ref[0, :]
        m = jnp.mean(w)
        var = jnp.sum((w - m) ** 2) * (1.0 / (_D - 1))
        wn = (w - m) * jax.lax.rsqrt(var + 1e-5)
        ww = 1.0 / (1.0 + jnp.exp(-7.0 * wn))
        ww_ref[0, :] = ww
        ew = e_ref[:, :] * ww[None, :]
        ew2_ref[:, :] = (ew * (2.0 * s)).astype(jnp.bfloat16)
        ones = jnp.ones((1, _D), dtype=jnp.float32)
        ees_ref[0, :] = jax.lax.dot_general(
            ones, ew * ew, (((1,), (1,)), ((), ())),
            preferred_element_type=jnp.float32)[0, :] * s
        arelu_ref[:, :] = jnp.zeros((8, _C), jnp.float32)
        acnt_ref[:, :] = jnp.zeros((8, _C), jnp.float32)
        amk_ref[:, :] = jnp.zeros((8, 128), jnp.float32)
        axx_ref[:, :] = jnp.zeros((8, _D), jnp.float32)

    ww = ww_ref[0, :]
    xw = x_ref[:, :] * ww[None, :]
    m2 = jax.lax.dot_general(
        xw.astype(jnp.bfloat16), ew2_ref[:, :],
        (((1,), (1,)), ((), ())),
        preferred_element_type=jnp.float32) - ees_ref[0, :][None, :]

    cols = jax.lax.broadcasted_iota(jnp.int32, (_BM, _C), 1)
    eq = cols == k_ref[:, :]
    sel = jnp.where(eq, m2, 0.0)
    ones_c = jnp.ones((_C, 1), dtype=jnp.float32)
    mk = jax.lax.dot_general(
        sel, ones_c, (((1,), (0,)), ((), ())),
        preferred_element_type=jnp.float32)

    diff = m2 - mk
    cond = jnp.logical_or(diff >= 0.0, eq)
    arelu_ref[:, :] = arelu_ref[:, :] + _fold_rows(jnp.maximum(diff, 0.0))
    acnt_ref[:, :] = acnt_ref[:, :] + _fold_rows(jnp.where(cond, 1.0, 0.0))
    amk_ref[:, 0:1] = amk_ref[:, 0:1] + _fold_rows(mk)
    axx_ref[:, :] = axx_ref[:, :] + _fold_rows(xw * xw)

    @pl.when(i == _NB - 1)
    def _fin():
        sum_mk = jnp.sum(amk_ref[:, 0])
        sum_xx = jnp.sum(axx_ref[:, :]) * s
        out0_ref[0] = (sum_xx - sum_mk) * (1.0 / _B)
        out1_ref[0] = jnp.sum(arelu_ref[:, :]) / jnp.sum(acnt_ref[:, :])


def kernel(predict_latent, k, weight, embeds):
    k2 = k.astype(jnp.int32).reshape(_B, 1)
    w2 = weight.reshape(1, _D)
    out0, out1 = pl.pallas_call(
        _dist_kernel,
        grid=(_NB,),
        in_specs=[
            pl.BlockSpec((_BM, _D), lambda i: (i, 0)),
            pl.BlockSpec((_BM, 1), lambda i: (i, 0)),
            pl.BlockSpec((1, _D), lambda i: (0, 0)),
            pl.BlockSpec((_C, _D), lambda i: (0, 0)),
        ],
        out_specs=[
            pl.BlockSpec(memory_space=pltpu.SMEM),
            pl.BlockSpec(memory_space=pltpu.SMEM),
        ],
        out_shape=[
            jax.ShapeDtypeStruct((1,), jnp.float32),
            jax.ShapeDtypeStruct((1,), jnp.float32),
        ],
        scratch_shapes=[
            pltpu.VMEM((1, _D), jnp.float32),
            pltpu.VMEM((_C, _D), jnp.bfloat16),
            pltpu.VMEM((1, _C), jnp.float32),
            pltpu.VMEM((8, _C), jnp.float32),
            pltpu.VMEM((8, _C), jnp.float32),
            pltpu.VMEM((8, 128), jnp.float32),
            pltpu.VMEM((8, _D), jnp.float32),
        ],
    )(predict_latent, k2, w2, embeds)
    return (out0[0], out1[0])


# speedup vs baseline: 1.0566x; 1.0566x over previous
import jax
import jax.numpy as jnp
from jax.experimental import pallas as pl
from jax.experimental.pallas import tpu as pltpu

_B = 4096
_D = 128
_C = 512
_NB = 2
_BM = _B // _NB


def _dist_kernel(x_ref, k_ref, w_ref, e_ref, out0_ref, out1_ref,
                 ww_ref, ew2_ref, ees_ref, acc_ref):
    i = pl.program_id(0)
    s = 1.0 / _D
    ones_d = jnp.ones((_D, 1), dtype=jnp.float32)

    @pl.when(i == 0)
    def _prep():
        w = w_ref[0:1, :]
        sw = jax.lax.dot_general(
            w, ones_d, (((1,), (0,)), ((), ())),
            preferred_element_type=jnp.float32)
        sww = jax.lax.dot_general(
            w * w, ones_d, (((1,), (0,)), ((), ())),
            preferred_element_type=jnp.float32)
        mean = sw * s
        var = (sww - sw * mean) * (1.0 / (_D - 1))
        wn = (w - mean) * jax.lax.rsqrt(var + 1e-5)
        ww = 1.0 / (1.0 + jnp.exp(-7.0 * wn))
        ww_ref[0:1, :] = ww
        ew = e_ref[:, :] * ww
        ew2_ref[:, :] = (ew * (2.0 * s)).astype(jnp.bfloat16)
        ones = jnp.ones((1, _D), dtype=jnp.float32)
        ees_ref[0, :] = jax.lax.dot_general(
            ones, ew * ew, (((1,), (1,)), ((), ())),
            preferred_element_type=jnp.float32)[0, :] * s
        acc_ref[0] = 0.0
        acc_ref[1] = 0.0
        acc_ref[2] = 0.0
        acc_ref[3] = 0.0

    ww = ww_ref[0, :]
    xw = x_ref[:, :] * ww[None, :]
    m2 = jax.lax.dot_general(
        xw.astype(jnp.bfloat16), ew2_ref[:, :],
        (((1,), (1,)), ((), ())),
        preferred_element_type=jnp.float32) - ees_ref[0, :][None, :]

    cols = jax.lax.broadcasted_iota(jnp.int32, (_BM, _C), 1)
    eq = cols == k_ref[:, :]
    sel = jnp.where(eq, m2, 0.0)
    ones_c = jnp.ones((_C, 1), dtype=jnp.float32)
    mk = jax.lax.dot_general(
        sel, ones_c, (((1,), (0,)), ((), ())),
        preferred_element_type=jnp.float32)

    diff = m2 - mk
    cond = jnp.logical_or(diff >= 0.0, eq)
    acc_ref[0] = acc_ref[0] + jnp.sum(jnp.maximum(diff, 0.0))
    acc_ref[1] = acc_ref[1] + jnp.sum(cond.astype(jnp.float32))
    acc_ref[2] = acc_ref[2] + jnp.sum(mk)
    acc_ref[3] = acc_ref[3] + jnp.sum(xw * xw)

    @pl.when(i == _NB - 1)
    def _fin():
        out0_ref[0] = (acc_ref[3] * s - acc_ref[2]) * (1.0 / _B)
        out1_ref[0] = acc_ref[0] / acc_ref[1]


def kernel(predict_latent, k, weight, embeds):
    k2 = k.astype(jnp.int32).reshape(_B, 1)
    w2 = weight.reshape(1, _D)
    out0, out1 = pl.pallas_call(
        _dist_kernel,
        grid=(_NB,),
        in_specs=[
            pl.BlockSpec((_BM, _D), lambda i: (i, 0)),
            pl.BlockSpec((_BM, 1), lambda i: (i, 0)),
            pl.BlockSpec((1, _D), lambda i: (0, 0)),
            pl.BlockSpec((_C, _D), lambda i: (0, 0)),
        ],
        out_specs=[
            pl.BlockSpec(memory_space=pltpu.SMEM),
            pl.BlockSpec(memory_space=pltpu.SMEM),
        ],
        out_shape=[
            jax.ShapeDtypeStruct((1,), jnp.float32),
            jax.ShapeDtypeStruct((1,), jnp.float32),
        ],
        scratch_shapes=[
            pltpu.VMEM((1, _D), jnp.float32),
            pltpu.VMEM((_C, _D), jnp.bfloat16),
            pltpu.VMEM((1, _C), jnp.float32),
            pltpu.SMEM((4,), jnp.float32),
        ],
    )(predict_latent, k2, w2, embeds)
    return (out0[0], out1[0])
